# Initial kernel scaffold; baseline (speedup 1.0000x reference)
import jax
import jax.numpy as jnp
from jax.experimental import pallas as pl


def kernel(x, edge_index, edge_attr, u, batch, W1, b1, W2, b2):
    raise NotImplementedError("write your pallas kernel here")


# speedup vs baseline: 7.2250x; 7.2250x over previous
import functools

import jax
import jax.numpy as jnp
from jax import lax
from jax.experimental import pallas as pl
from jax.experimental.pallas import tpu as pltpu
from jax.experimental.pallas import tpu_sc as plsc

N_NODES = 10000
N_EDGES = 320000
H = 128

NC = 2
NS = 16
NW = NC * NS
EPW = N_EDGES // NW
B = 125
NB = EPW // B
RPT = N_NODES // NS
ZR = 25
assert EPW % B == 0 and NB % 2 == 0 and RPT % ZR == 0


def _sc_scatter_partials(col2d, edge_attr):
    mesh = plsc.VectorSubcoreMesh(core_axis_name="c", subcore_axis_name="s")

    @functools.partial(
        pl.kernel,
        out_type=jax.ShapeDtypeStruct((NC, N_NODES, H), jnp.float32),
        mesh=mesh,
        scratch_types=[
            pltpu.VMEM((NB, B), jnp.int32),
            pltpu.VMEM((2, B, H), jnp.float32),
            pltpu.VMEM((ZR, H), jnp.float32),
            pltpu.VMEM_SHARED((N_NODES, H), jnp.float32),
            pltpu.SemaphoreType.DMA,
            pltpu.SemaphoreType.DMA,
        ],
        compiler_params=pltpu.CompilerParams(use_tc_tiling_on_sc=False),
    )
    def k(col_hbm, ea_hbm, out_hbm, idx_v, rows_v, zbuf, agg_sh, sem0, sem1):
        cid = lax.axis_index("c")
        sid = lax.axis_index("s")
        wid = cid * NS + sid
        erow = wid * NB
        ebase = wid * EPW
        sems = (sem0, sem1)

        pltpu.sync_copy(col_hbm.at[pl.ds(erow, NB)], idx_v)

        zv = jnp.zeros((16,), jnp.float32)

        def _zrow(i, _):
            for j in range(H // 16):
                zbuf[i, pl.ds(j * 16, 16)] = zv
            return 0

        lax.fori_loop(0, ZR, _zrow, 0)
        for z in range(RPT // ZR):
            pltpu.sync_copy(zbuf, agg_sh.at[pl.ds(sid * RPT + z * ZR, ZR)])
        plsc.subcore_barrier()

        for b in range(2):
            pltpu.async_copy(
                ea_hbm.at[pl.ds(ebase + b * B, B)], rows_v.at[b], sems[b])

        def _step(p, _):
            for b in range(2):
                chunk = 2 * p + b
                pltpu.make_async_copy(
                    ea_hbm.at[pl.ds(ebase + chunk * B, B)],
                    rows_v.at[b], sems[b]).wait()
                pltpu.sync_copy(rows_v.at[b], agg_sh.at[idx_v.at[chunk]],
                                add=True)

                @pl.when(p < NB // 2 - 1)
                def _():
                    pltpu.async_copy(
                        ea_hbm.at[pl.ds(ebase + (chunk + 2) * B, B)],
                        rows_v.at[b], sems[b])
            return 0

        lax.fori_loop(0, NB // 2, _step, 0)
        plsc.subcore_barrier()

        pltpu.sync_copy(agg_sh.at[pl.ds(sid * RPT, RPT)],
                        out_hbm.at[cid].at[pl.ds(sid * RPT, RPT)])

    return k(col2d, edge_attr)


def _mlp_body(x_ref, p0_ref, p1_ref, w1x_ref, w1a_ref, w2_ref, b1_ref,
              b2_ref, out_ref):
    agg = p0_ref[...] + p1_ref[...]
    h = (jnp.dot(x_ref[...], w1x_ref[...], preferred_element_type=jnp.float32)
         + jnp.dot(agg, w1a_ref[...], preferred_element_type=jnp.float32)
         + b1_ref[...])
    h = jnp.maximum(h, 0.0)
    out_ref[...] = (jnp.dot(h, w2_ref[...],
                            preferred_element_type=jnp.float32) + b2_ref[...])


def _mlp(x, p0, p1, w1x_t, w1a_t, w2_t, b1, b2):
    blk = 2000
    grid = (N_NODES // blk,)
    row_spec = pl.BlockSpec((blk, H), lambda i: (i, 0))
    full = pl.BlockSpec((H, H), lambda i: (0, 0))
    vec = pl.BlockSpec((1, H), lambda i: (0, 0))
    return pl.pallas_call(
        _mlp_body,
        grid=grid,
        in_specs=[row_spec, row_spec, row_spec, full, full, full, vec, vec],
        out_specs=row_spec,
        out_shape=jax.ShapeDtypeStruct((N_NODES, H), jnp.float32),
    )(x, p0, p1, w1x_t, w1a_t, w2_t, b1, b2)


def kernel(x, edge_index, edge_attr, u, batch, W1, b1, W2, b2):
    del u, batch
    col2d = edge_index[1].reshape(N_EDGES // B, B)
    partials = _sc_scatter_partials(col2d, edge_attr)
    w1t = W1.T
    return _mlp(x, partials[0], partials[1], w1t[:H], w1t[H:], W2.T,
                b1.reshape(1, H), b2.reshape(1, H))
